# Initial kernel scaffold; baseline (speedup 1.0000x reference)
import jax
import jax.numpy as jnp
from jax.experimental import pallas as pl


def kernel(input_features, doc_ids, loss_mask, xl_k_hat, xl_v, xl_z, xl_doc_ids, pos_offset, ln_g, ln_b, W_q, W_k, W_v, W_g, W_res, x_u, codebook):
    raise NotImplementedError("write your pallas kernel here")


# speedup vs baseline: 1.8403x; 1.8403x over previous
import jax
import jax.numpy as jnp
from jax.experimental import pallas as pl

_B, _S, _D, _H, _DK, _DV, _M, _C = 1, 2048, 1024, 16, 64, 64, 256, 512
_SB = 512


def _proj_body(x_ref, g_ref, b_ref, w_ref, xu_ref, sh_ref, sht_ref, out_ref):
    x = x_ref[...]
    mu = jnp.mean(x, axis=1, keepdims=True)
    var = jnp.mean((x - mu) ** 2, axis=1, keepdims=True)
    xt = (x - mu) * jax.lax.rsqrt(var + 1e-6) * g_ref[...] + b_ref[...]
    y = jnp.dot(xt, w_ref[...], preferred_element_type=jnp.float32)
    yqk = y[:, : 2 * _D]
    s1 = jnp.dot(yqk, sh_ref[...], preferred_element_type=jnp.float32, precision=jax.lax.Precision.HIGHEST)
    s2 = jnp.dot(yqk * yqk, sh_ref[...], preferred_element_type=jnp.float32, precision=jax.lax.Precision.HIGHEST)
    mu_h = s1 * (1.0 / _DK)
    var_h = s2 * (1.0 / _DK) - mu_h * mu_h
    rstd_h = jax.lax.rsqrt(var_h + 1e-6)
    mu_b = jnp.dot(mu_h, sht_ref[...], preferred_element_type=jnp.float32, precision=jax.lax.Precision.HIGHEST)
    rs_b = jnp.dot(rstd_h, sht_ref[...], preferred_element_type=jnp.float32, precision=jax.lax.Precision.HIGHEST)
    qk = (yqk - mu_b) * rs_b
    q = qk[:, :_D] + xu_ref[...]
    k = qk[:, _D:]
    v = y[:, 2 * _D : 3 * _D]
    gg = y[:, 3 * _D :]
    ga = gg * jax.nn.sigmoid(gg)
    out_ref[...] = jnp.concatenate([q, k, v, ga], axis=1)


def _vq_body(k_ref, cbt_ref, cb_ref, lm_ref, khat_ref, loss_ref):
    h = pl.program_id(0)
    k = k_ref[...].reshape(_S, _DK)
    cbt = cbt_ref[...].reshape(_DK, _C)
    cb = cb_ref[...].reshape(_C, _DK)
    dot = jnp.dot(k, cbt, preferred_element_type=jnp.float32)
    c2 = jnp.sum(cbt * cbt, axis=0, keepdims=True)
    k2 = jnp.sum(k * k, axis=1, keepdims=True)
    dneg = k2 - 2.0 * dot + c2
    m = jnp.min(dneg, axis=1, keepdims=True)
    iota = jax.lax.broadcasted_iota(jnp.int32, (_S, _C), 1)
    idx = jnp.min(jnp.where(dneg == m, iota, _C), axis=1, keepdims=True)
    onehot = (iota == idx).astype(jnp.float32)
    khat = jnp.dot(onehot, cb, preferred_element_type=jnp.float32, precision=jax.lax.Precision.HIGHEST)
    khat_ref[...] = khat.reshape(1, _S, _DK)
    diff = k - khat
    d2r = jnp.sum(diff * diff, axis=1, keepdims=True)
    tot = jnp.sum(d2r * lm_ref[...], axis=0, keepdims=True)

    @pl.when(h == 0)
    def _():
        loss_ref[...] = jnp.zeros_like(loss_ref)

    loss_ref[...] += tot


def _attn_body(q_ref, xlk_ref, xlv_ref, khat_ref, v_ref, g_ref, out_ref):
    sblk = pl.program_id(1)
    q = q_ref[...].reshape(_SB, _DK)
    xlk = xlk_ref[...].reshape(_M, _DK)
    xlv = xlv_ref[...].reshape(_M, _DV)
    kh = khat_ref[...].reshape(_S, _DK)
    vv = v_ref[...].reshape(_S, _DV)
    inv_tau = 1.0 / (_DK**0.5)
    dn = (((1,), (1,)), ((), ()))
    sm = jax.lax.dot_general(q, xlk, dn, preferred_element_type=jnp.float32) * inv_tau
    ss = jax.lax.dot_general(q, kh, dn, preferred_element_type=jnp.float32) * inv_tau
    rows = jax.lax.broadcasted_iota(jnp.int32, (_SB, _S), 0) + sblk * _SB
    cols = jax.lax.broadcasted_iota(jnp.int32, (_SB, _S), 1)
    ss = jnp.where(cols <= rows, ss, -1e30)
    mx = jnp.maximum(
        jnp.max(sm, axis=1, keepdims=True), jnp.max(ss, axis=1, keepdims=True)
    )
    em = jnp.exp(sm - mx)
    es = jnp.exp(ss - mx)
    denom = jnp.sum(em, axis=1, keepdims=True) + jnp.sum(es, axis=1, keepdims=True)
    wv = (
        jnp.dot(em, xlv, preferred_element_type=jnp.float32)
        + jnp.dot(es, vv, preferred_element_type=jnp.float32)
    ) / denom
    out_ref[...] = (wv * g_ref[...].reshape(_SB, _DV)).reshape(1, _SB, _DV)


def _out_body(wg_ref, w_ref, out_ref):
    out_ref[...] = jnp.dot(wg_ref[...], w_ref[...], preferred_element_type=jnp.float32)


def kernel(input_features, doc_ids, loss_mask, xl_k_hat, xl_v, xl_z, xl_doc_ids,
           pos_offset, ln_g, ln_b, W_q, W_k, W_v, W_g, W_res, x_u, codebook):
    f32 = jnp.float32
    x = input_features.reshape(_S, _D)
    w_all = jnp.concatenate([W_q, W_k, W_v, W_g], axis=1)
    xu_flat = x_u.reshape(1, _H * _DK)
    heads = jnp.arange(2 * _H, dtype=jnp.int32)
    sh = (jnp.arange(2 * _D, dtype=jnp.int32)[:, None] // _DK == heads[None, :]).astype(f32)
    sht = sh.T

    nsb = _S // 256
    proj = pl.pallas_call(
        _proj_body,
        grid=(nsb,),
        in_specs=[
            pl.BlockSpec((256, _D), lambda i: (i, 0)),
            pl.BlockSpec((1, _D), lambda i: (0, 0)),
            pl.BlockSpec((1, _D), lambda i: (0, 0)),
            pl.BlockSpec((_D, 4 * _D), lambda i: (0, 0)),
            pl.BlockSpec((1, _D), lambda i: (0, 0)),
            pl.BlockSpec((2 * _D, 2 * _H), lambda i: (0, 0)),
            pl.BlockSpec((2 * _H, 2 * _D), lambda i: (0, 0)),
        ],
        out_specs=pl.BlockSpec((256, 4 * _D), lambda i: (i, 0)),
        out_shape=jax.ShapeDtypeStruct((_S, 4 * _D), f32),
    )(x, ln_g.reshape(1, _D), ln_b.reshape(1, _D), w_all, xu_flat, sh, sht)

    t = proj.reshape(_S, 4 * _H, _DK).transpose(1, 0, 2)
    q3 = t[:_H]
    k3 = t[_H : 2 * _H]
    v3 = t[2 * _H : 3 * _H]
    g3 = t[3 * _H :]

    cbt = codebook.transpose(0, 2, 1)
    khat3, loss = pl.pallas_call(
        _vq_body,
        grid=(_H,),
        in_specs=[
            pl.BlockSpec((1, _S, _DK), lambda h: (h, 0, 0)),
            pl.BlockSpec((1, _DK, _C), lambda h: (h, 0, 0)),
            pl.BlockSpec((1, _C, _DK), lambda h: (h, 0, 0)),
            pl.BlockSpec((_S, 1), lambda h: (0, 0)),
        ],
        out_specs=[
            pl.BlockSpec((1, _S, _DK), lambda h: (h, 0, 0)),
            pl.BlockSpec((1, 1), lambda h: (0, 0)),
        ],
        out_shape=[
            jax.ShapeDtypeStruct((_H, _S, _DK), f32),
            jax.ShapeDtypeStruct((1, 1), f32),
        ],
    )(k3, cbt, codebook, loss_mask.reshape(_S, 1))

    wv3 = pl.pallas_call(
        _attn_body,
        grid=(_H, _S // _SB),
        in_specs=[
            pl.BlockSpec((1, _SB, _DK), lambda h, s: (h, s, 0)),
            pl.BlockSpec((1, 1, _M, _DK), lambda h, s: (0, h, 0, 0)),
            pl.BlockSpec((1, 1, _M, _DV), lambda h, s: (0, h, 0, 0)),
            pl.BlockSpec((1, _S, _DK), lambda h, s: (h, 0, 0)),
            pl.BlockSpec((1, _S, _DV), lambda h, s: (h, 0, 0)),
            pl.BlockSpec((1, _SB, _DV), lambda h, s: (h, s, 0)),
        ],
        out_specs=pl.BlockSpec((1, _SB, _DV), lambda h, s: (h, s, 0)),
        out_shape=jax.ShapeDtypeStruct((_H, _S, _DV), f32),
    )(q3, xl_k_hat, xl_v, khat3, v3, g3)

    wg = wv3.transpose(1, 0, 2).reshape(_S, _H * _DV)

    res = pl.pallas_call(
        _out_body,
        grid=(nsb,),
        in_specs=[
            pl.BlockSpec((256, _D), lambda i: (i, 0)),
            pl.BlockSpec((_D, _D), lambda i: (0, 0)),
        ],
        out_specs=pl.BlockSpec((256, _D), lambda i: (i, 0)),
        out_shape=jax.ShapeDtypeStruct((_S, _D), f32),
    )(wg, W_res)

    denom = jnp.sum(loss_mask) * (_H * _DK) + 1e-8
    l = (loss[0, 0] / denom).astype(f32)
    return res.reshape(_B, _S, _D), l, l


# speedup vs baseline: 2.1952x; 1.1929x over previous
import jax
import jax.numpy as jnp
from jax.experimental import pallas as pl

_B, _S, _D, _H, _DK, _DV, _M, _C = 1, 2048, 1024, 16, 64, 64, 256, 512
_SB = 512


def _hn_rows(x):
    mu = jnp.mean(x, axis=-1, keepdims=True)
    var = jnp.mean((x - mu) ** 2, axis=-1, keepdims=True)
    return (x - mu) * jax.lax.rsqrt(var + 1e-6)


def _proj_body(x_ref, g_ref, b_ref, w_ref, out_ref):
    x = x_ref[...]
    mu = jnp.mean(x, axis=1, keepdims=True)
    var = jnp.mean((x - mu) ** 2, axis=1, keepdims=True)
    xt = (x - mu) * jax.lax.rsqrt(var + 1e-6) * g_ref[...] + b_ref[...]
    y = jnp.dot(xt, w_ref[...], preferred_element_type=jnp.float32)
    gg = y[:, 3 * _D :]
    ga = gg * jax.nn.sigmoid(gg)
    out_ref[...] = jnp.concatenate([y[:, : 3 * _D], ga], axis=1)


def _vq_body(k_ref, cbt_ref, cb_ref, lm_ref, khat_ref, loss_ref):
    h = pl.program_id(0)
    k = _hn_rows(k_ref[...].reshape(_S, _DK))
    cbt = cbt_ref[...].reshape(_DK, _C)
    cb = cb_ref[...].reshape(_C, _DK)
    dot = jnp.dot(k, cbt, preferred_element_type=jnp.float32)
    c2 = jnp.sum(cbt * cbt, axis=0, keepdims=True)
    k2 = jnp.sum(k * k, axis=1, keepdims=True)
    dneg = k2 - 2.0 * dot + c2
    m = jnp.min(dneg, axis=1, keepdims=True)
    iota = jax.lax.broadcasted_iota(jnp.int32, (_S, _C), 1)
    idx = jnp.min(jnp.where(dneg == m, iota, _C), axis=1, keepdims=True)
    onehot = (iota == idx).astype(jnp.float32)
    khat = jnp.dot(onehot, cb, preferred_element_type=jnp.float32)
    khat_ref[...] = khat.reshape(1, _S, _DK)
    diff = k - khat
    d2r = jnp.sum(diff * diff, axis=1, keepdims=True)
    tot = jnp.sum(d2r * lm_ref[...], axis=0, keepdims=True)

    @pl.when(h == 0)
    def _():
        loss_ref[...] = jnp.zeros_like(loss_ref)

    loss_ref[...] += tot


def _attn_body(q_ref, xu_ref, xlk_ref, xlv_ref, khat_ref, v_ref, g_ref, out_ref):
    sblk = pl.program_id(1)
    q = _hn_rows(q_ref[...].reshape(_SB, _DK)) + xu_ref[...].reshape(1, _DK)
    xlk = xlk_ref[...].reshape(_M, _DK)
    xlv = xlv_ref[...].reshape(_M, _DV)
    kh = khat_ref[...].reshape(_S, _DK)
    vv = v_ref[...].reshape(_S, _DV)
    inv_tau = 1.0 / (_DK**0.5)
    dn = (((1,), (1,)), ((), ()))
    sm = jax.lax.dot_general(q, xlk, dn, preferred_element_type=jnp.float32) * inv_tau
    ss = jax.lax.dot_general(q, kh, dn, preferred_element_type=jnp.float32) * inv_tau
    rows = jax.lax.broadcasted_iota(jnp.int32, (_SB, _S), 0) + sblk * _SB
    cols = jax.lax.broadcasted_iota(jnp.int32, (_SB, _S), 1)
    ss = jnp.where(cols <= rows, ss, -1e30)
    mx = jnp.maximum(
        jnp.max(sm, axis=1, keepdims=True), jnp.max(ss, axis=1, keepdims=True)
    )
    em = jnp.exp(sm - mx)
    es = jnp.exp(ss - mx)
    denom = jnp.sum(em, axis=1, keepdims=True) + jnp.sum(es, axis=1, keepdims=True)
    wv = (
        jnp.dot(em, xlv, preferred_element_type=jnp.float32)
        + jnp.dot(es, vv, preferred_element_type=jnp.float32)
    ) / denom
    out_ref[...] = (wv * g_ref[...].reshape(_SB, _DV)).reshape(1, _SB, _DV)


def _out_body(wg_ref, w_ref, out_ref):
    out_ref[...] = jnp.dot(wg_ref[...], w_ref[...], preferred_element_type=jnp.float32)


def kernel(input_features, doc_ids, loss_mask, xl_k_hat, xl_v, xl_z, xl_doc_ids,
           pos_offset, ln_g, ln_b, W_q, W_k, W_v, W_g, W_res, x_u, codebook):
    f32 = jnp.float32
    x = input_features.reshape(_S, _D)
    w_all = jnp.concatenate([W_q, W_k, W_v, W_g], axis=1)

    nsb = _S // 256
    proj = pl.pallas_call(
        _proj_body,
        grid=(nsb,),
        in_specs=[
            pl.BlockSpec((256, _D), lambda i: (i, 0)),
            pl.BlockSpec((1, _D), lambda i: (0, 0)),
            pl.BlockSpec((1, _D), lambda i: (0, 0)),
            pl.BlockSpec((_D, 4 * _D), lambda i: (0, 0)),
        ],
        out_specs=pl.BlockSpec((256, 4 * _D), lambda i: (i, 0)),
        out_shape=jax.ShapeDtypeStruct((_S, 4 * _D), f32),
    )(x, ln_g.reshape(1, _D), ln_b.reshape(1, _D), w_all)

    t = proj.reshape(_S, 4 * _H, _DK).transpose(1, 0, 2)
    q3 = t[:_H]
    k3 = t[_H : 2 * _H]
    v3 = t[2 * _H : 3 * _H]
    g3 = t[3 * _H :]

    cbt = codebook.transpose(0, 2, 1)
    khat3, loss = pl.pallas_call(
        _vq_body,
        grid=(_H,),
        in_specs=[
            pl.BlockSpec((1, _S, _DK), lambda h: (h, 0, 0)),
            pl.BlockSpec((1, _DK, _C), lambda h: (h, 0, 0)),
            pl.BlockSpec((1, _C, _DK), lambda h: (h, 0, 0)),
            pl.BlockSpec((_S, 1), lambda h: (0, 0)),
        ],
        out_specs=[
            pl.BlockSpec((1, _S, _DK), lambda h: (h, 0, 0)),
            pl.BlockSpec((1, 1), lambda h: (0, 0)),
        ],
        out_shape=[
            jax.ShapeDtypeStruct((_H, _S, _DK), f32),
            jax.ShapeDtypeStruct((1, 1), f32),
        ],
    )(k3, cbt, codebook, loss_mask.reshape(_S, 1))

    wv3 = pl.pallas_call(
        _attn_body,
        grid=(_H, _S // _SB),
        in_specs=[
            pl.BlockSpec((1, _SB, _DK), lambda h, s: (h, s, 0)),
            pl.BlockSpec((1, 1, _DK), lambda h, s: (h, 0, 0)),
            pl.BlockSpec((1, 1, _M, _DK), lambda h, s: (0, h, 0, 0)),
            pl.BlockSpec((1, 1, _M, _DV), lambda h, s: (0, h, 0, 0)),
            pl.BlockSpec((1, _S, _DK), lambda h, s: (h, 0, 0)),
            pl.BlockSpec((1, _S, _DV), lambda h, s: (h, 0, 0)),
            pl.BlockSpec((1, _SB, _DV), lambda h, s: (h, s, 0)),
        ],
        out_specs=pl.BlockSpec((1, _SB, _DV), lambda h, s: (h, s, 0)),
        out_shape=jax.ShapeDtypeStruct((_H, _S, _DV), f32),
    )(q3, x_u.reshape(_H, 1, _DK), xl_k_hat, xl_v, khat3, v3, g3)

    wg = wv3.transpose(1, 0, 2).reshape(_S, _H * _DV)

    res = pl.pallas_call(
        _out_body,
        grid=(nsb,),
        in_specs=[
            pl.BlockSpec((256, _D), lambda i: (i, 0)),
            pl.BlockSpec((_D, _D), lambda i: (0, 0)),
        ],
        out_specs=pl.BlockSpec((256, _D), lambda i: (i, 0)),
        out_shape=jax.ShapeDtypeStruct((_S, _D), f32),
    )(wg, W_res)

    denom = jnp.sum(loss_mask) * (_H * _DK) + 1e-8
    l = (loss[0, 0] / denom).astype(f32)
    return res.reshape(_B, _S, _D), l, l


# speedup vs baseline: 4.3227x; 1.9691x over previous
import jax
import jax.numpy as jnp
from jax.experimental import pallas as pl

_B, _S, _D, _H, _DK, _DV, _M, _C = 1, 2048, 1024, 16, 64, 64, 256, 512
_SB = 1024


def _proj_body(x_ref, g_ref, b_ref, w_ref, out_ref):
    x = x_ref[...]
    mu = jnp.mean(x, axis=1, keepdims=True)
    var = jnp.mean((x - mu) ** 2, axis=1, keepdims=True)
    xt = (x - mu) * jax.lax.rsqrt(var + 1e-6) * g_ref[...] + b_ref[...]
    yt = jax.lax.dot_general(
        w_ref[...], xt, (((0,), (1,)), ((), ())), preferred_element_type=jnp.float32
    )
    gg = yt[3 * _D :, :]
    ga = gg * jax.nn.sigmoid(gg)
    out_ref[...] = jnp.concatenate([yt[: 3 * _D, :], ga], axis=0)


def _hn_cols(x):
    mu = jnp.mean(x, axis=0, keepdims=True)
    var = jnp.mean((x - mu) ** 2, axis=0, keepdims=True)
    return (x - mu) * jax.lax.rsqrt(var + 1e-6)


def _vq_body(kt_ref, cb_ref, lm_ref, khat_ref, loss_ref):
    h = pl.program_id(0)
    kt = _hn_cols(kt_ref[...])
    cb = cb_ref[...].reshape(_C, _DK)
    dt = jax.lax.dot_general(
        cb, kt, (((1,), (0,)), ((), ())), preferred_element_type=jnp.float32
    )
    c2 = jnp.sum(cb * cb, axis=1, keepdims=True)
    k2 = jnp.sum(kt * kt, axis=0, keepdims=True)
    dneg = k2 - 2.0 * dt + c2
    m = jnp.min(dneg, axis=0, keepdims=True)
    iota = jax.lax.broadcasted_iota(jnp.int32, (_C, _S), 0)
    idx = jnp.min(jnp.where(dneg == m, iota, _C), axis=0, keepdims=True)
    onehot = (iota == idx).astype(jnp.float32)
    khat = jax.lax.dot_general(
        cb, onehot, (((0,), (0,)), ((), ())), preferred_element_type=jnp.float32
    )
    khat_ref[...] = khat.reshape(1, _DK, _S)
    diff = kt - khat
    d2c = jnp.sum(diff * diff, axis=0, keepdims=True)
    tot = jax.lax.dot_general(
        d2c, lm_ref[...], (((1,), (1,)), ((), ())),
        preferred_element_type=jnp.float32,
        precision=jax.lax.Precision.HIGHEST,
    )

    @pl.when(h == 0)
    def _():
        loss_ref[...] = jnp.zeros_like(loss_ref)

    loss_ref[...] += tot


def _make_attn_body(kself):
    def _attn_body(pen_ref, qt_ref, xu_ref, xlk_ref, xlv_ref, kht_ref, vt_ref,
                   gt_ref, out_ref):
        qt = _hn_cols(qt_ref[...]) + xu_ref[...].reshape(_DK, 1)
        xlk = xlk_ref[...].reshape(_M, _DK)
        xlv = xlv_ref[...].reshape(_M, _DV)
        kht = kht_ref[...].reshape(_DK, kself)
        vt = vt_ref[...]
        inv_tau = 1.0 / (_DK**0.5)
        smt = jax.lax.dot_general(
            xlk, qt, (((1,), (0,)), ((), ())), preferred_element_type=jnp.float32
        ) * inv_tau
        sst = jax.lax.dot_general(
            kht, qt, (((0,), (0,)), ((), ())), preferred_element_type=jnp.float32
        ) * inv_tau
        sst = sst + pen_ref[...]
        mxt = jnp.maximum(
            jnp.max(smt, axis=0, keepdims=True), jnp.max(sst, axis=0, keepdims=True)
        )
        emt = jnp.exp(smt - mxt)
        est = jnp.exp(sst - mxt)
        dn_t = jnp.sum(emt, axis=0, keepdims=True) + jnp.sum(est, axis=0, keepdims=True)
        wvt = jax.lax.dot_general(
            xlv, emt, (((0,), (0,)), ((), ())), preferred_element_type=jnp.float32
        ) + jax.lax.dot_general(
            vt, est, (((1,), (0,)), ((), ())), preferred_element_type=jnp.float32
        )
        out_ref[...] = ((wvt / dn_t) * gt_ref[...]).reshape(1, _DV, _SB)

    return _attn_body


def _out_body(wg_ref, w_ref, out_ref):
    wgt = jnp.concatenate([wg_ref[hh] for hh in range(_H)], axis=0)
    out_ref[...] = jax.lax.dot_general(
        wgt, w_ref[...], (((0,), (0,)), ((), ())),
        preferred_element_type=jnp.float32,
    )


def kernel(input_features, doc_ids, loss_mask, xl_k_hat, xl_v, xl_z, xl_doc_ids,
           pos_offset, ln_g, ln_b, W_q, W_k, W_v, W_g, W_res, x_u, codebook):
    f32 = jnp.float32
    x = input_features.reshape(_S, _D)
    w_all = jnp.concatenate([W_q, W_k, W_v, W_g], axis=1)

    nsb = _S // 256
    yt = pl.pallas_call(
        _proj_body,
        grid=(nsb,),
        in_specs=[
            pl.BlockSpec((256, _D), lambda i: (i, 0)),
            pl.BlockSpec((1, _D), lambda i: (0, 0)),
            pl.BlockSpec((1, _D), lambda i: (0, 0)),
            pl.BlockSpec((_D, 4 * _D), lambda i: (0, 0)),
        ],
        out_specs=pl.BlockSpec((4 * _D, 256), lambda i: (0, i)),
        out_shape=jax.ShapeDtypeStruct((4 * _D, _S), f32),
    )(x, ln_g.reshape(1, _D), ln_b.reshape(1, _D), w_all)

    khat3, loss = pl.pallas_call(
        _vq_body,
        grid=(_H,),
        in_specs=[
            pl.BlockSpec((_DK, _S), lambda h: (_H + h, 0)),
            pl.BlockSpec((1, _C, _DK), lambda h: (h, 0, 0)),
            pl.BlockSpec((1, _S), lambda h: (0, 0)),
        ],
        out_specs=[
            pl.BlockSpec((1, _DK, _S), lambda h: (h, 0, 0)),
            pl.BlockSpec((1, 1), lambda h: (0, 0)),
        ],
        out_shape=[
            jax.ShapeDtypeStruct((_H, _DK, _S), f32),
            jax.ShapeDtypeStruct((1, 1), f32),
        ],
    )(yt, codebook, loss_mask.reshape(1, _S))

    xu3 = x_u.reshape(_H, _DK, 1)
    rows_i = jnp.arange(_SB, dtype=jnp.int32)[:, None]
    wv_parts = []
    for sidx in range(_S // _SB):
        kself = (sidx + 1) * _SB
        cols_i = jnp.arange(kself, dtype=jnp.int32)[:, None]
        pen = jnp.where(cols_i <= sidx * _SB + rows_i.T, 0.0, -1e30).astype(f32)
        wv_parts.append(
            pl.pallas_call(
                _make_attn_body(kself),
                grid=(_H,),
                in_specs=[
                    pl.BlockSpec((kself, _SB), lambda h: (0, 0)),
                    pl.BlockSpec((_DK, _SB), lambda h, _s=sidx: (h, _s)),
                    pl.BlockSpec((1, _DK, 1), lambda h: (h, 0, 0)),
                    pl.BlockSpec((1, 1, _M, _DK), lambda h: (0, h, 0, 0)),
                    pl.BlockSpec((1, 1, _M, _DV), lambda h: (0, h, 0, 0)),
                    pl.BlockSpec((1, _DK, kself), lambda h: (h, 0, 0)),
                    pl.BlockSpec((_DV, kself), lambda h: (2 * _H + h, 0)),
                    pl.BlockSpec((_DV, _SB), lambda h, _s=sidx: (3 * _H + h, _s)),
                ],
                out_specs=pl.BlockSpec((1, _DV, _SB), lambda h: (h, 0, 0)),
                out_shape=jax.ShapeDtypeStruct((_H, _DV, _SB), f32),
            )(pen, yt, xu3, xl_k_hat, xl_v, khat3, yt, yt)
        )
    wvg = jnp.concatenate(wv_parts, axis=2)

    res = pl.pallas_call(
        _out_body,
        grid=(nsb,),
        in_specs=[
            pl.BlockSpec((_H, _DV, 256), lambda i: (0, 0, i)),
            pl.BlockSpec((_D, _D), lambda i: (0, 0)),
        ],
        out_specs=pl.BlockSpec((256, _D), lambda i: (i, 0)),
        out_shape=jax.ShapeDtypeStruct((_S, _D), f32),
    )(wvg, W_res)

    denom = jnp.sum(loss_mask) * (_H * _DK) + 1e-8
    l = (loss[0, 0] / denom).astype(f32)
    return res.reshape(_B, _S, _D), l, l
